# baseline (speedup 1.0000x reference)
import jax
import jax.numpy as jnp
from jax import lax
from jax.experimental import pallas as pl
from jax.experimental.pallas import tpu as pltpu
from jax.experimental.pallas import tpu_sc as plsc

B, C, H, W = 4, 128, 512, 512
N = 64
HB = WB = 32
P = HB * WB
NPATCH = B * N
NW = 32
L = 16
PC = 64
NPC = P // PC


def _floor_vec(x):
    xi = x.astype(jnp.int32)
    xf = xi.astype(jnp.float32)
    neg = x < xf
    return jnp.where(neg, xi - 1, xi), jnp.where(neg, xf - 1.0, xf)


NPB = N
PPW = NPB // NW


def _tc_transpose():
    def body(i_ref, o_ref):
        o_ref[...] = i_ref[...].T

    return pl.pallas_call(
        body,
        grid=(H * W // 4096,),
        in_specs=[pl.BlockSpec((C, 4096), lambda i: (0, i))],
        out_specs=pl.BlockSpec((4096, C), lambda i: (i, 0)),
        out_shape=jax.ShapeDtypeStruct((H * W, C), jnp.float32),
    )


def _body(table_hbm, csu_hbm, out_hbm, csu_v,
          w00_v, w01_v, w10_v, w11_v, i00_v, i01_v, i10_v, i11_v,
          ra_bufs, rb_bufs, obuf_v, sem_a, sem_b):
    wid = lax.axis_index("s") * 2 + lax.axis_index("c")
    pltpu.sync_copy(csu_hbm, csu_v)
    idx_refs = (i00_v, i01_v, i10_v, i11_v)

    def fire(pc, bufs, sem):
        isl = pl.ds(pc * PC, PC)
        for iv, rv in zip(idx_refs, bufs):
            pltpu.async_copy(table_hbm.at[iv.at[isl]], rv, sem)

    def drain(pc, bufs, sem):
        isl = pl.ds(pc * PC, PC)
        for iv, rv in zip(idx_refs, bufs):
            pltpu.make_async_copy(table_hbm.at[iv.at[isl]], rv, sem).wait()

    def per_patch(t, _):
        patch = wid * PPW + t
        csrow = csu_v[patch, :]
        cs = csrow[0]
        sn = csrow[1]
        u = csrow[2]
        v = csrow[3]
        bhw = 0

        def grid_math(pg, _):
            pid = pg * L + lax.broadcasted_iota(jnp.int32, (L,), 0)
            hb = pid >> 5
            wb = pid & 31
            gu = (31 - hb).astype(jnp.float32)
            gv = (wb - 16).astype(jnp.float32)
            x = u + cs * gu - sn * gv
            y = v + sn * gu + cs * gv
            gx = (x + 0.5) * (2.0 / W) - 1.0
            gy = (y + 0.5) * (2.0 / H) - 1.0
            pv = (jnp.abs(gx) < 1.0) & (jnp.abs(gy) < 1.0)
            x0i, x0f = _floor_vec(x)
            y0i, y0f = _floor_vec(y)
            wx1 = x - x0f
            wx0 = 1.0 - wx1
            wy1 = y - y0f
            wy0 = 1.0 - wy1
            vx0 = (x0f >= 0.0) & (x0f <= W - 1.0)
            vx1 = (x0f + 1.0 >= 0.0) & (x0f + 1.0 <= W - 1.0)
            vy0 = (y0f >= 0.0) & (y0f <= H - 1.0)
            vy1 = (y0f + 1.0 >= 0.0) & (y0f + 1.0 <= H - 1.0)
            x0c = jnp.clip(x0i, 0, W - 1)
            x1c = jnp.clip(x0i + 1, 0, W - 1)
            y0r = jnp.clip(y0i, 0, H - 1) * W + bhw
            y1r = jnp.clip(y0i + 1, 0, H - 1) * W + bhw
            sl = pl.ds(pg * L, L)
            w00_v[sl] = jnp.where(vx0 & vy0 & pv, wx0 * wy0, 0.0)
            w01_v[sl] = jnp.where(vx1 & vy0 & pv, wx1 * wy0, 0.0)
            w10_v[sl] = jnp.where(vx0 & vy1 & pv, wx0 * wy1, 0.0)
            w11_v[sl] = jnp.where(vx1 & vy1 & pv, wx1 * wy1, 0.0)
            i00_v[sl] = y0r + x0c
            i01_v[sl] = y0r + x1c
            i10_v[sl] = y1r + x0c
            i11_v[sl] = y1r + x1c
            return 0

        lax.fori_loop(0, P // L, grid_math, 0)

        def consume(pc, bufs):
            r00_v, r01_v, r10_v, r11_v = bufs

            def combine(pg, _):
                psl = pl.ds(pc * PC + pg * L, L)
                w00 = w00_v[psl]
                w01 = w01_v[psl]
                w10 = w10_v[psl]
                w11 = w11_v[psl]
                for i in range(L):
                    p = pg * L + i
                    a00 = jnp.full((L,), w00[i])
                    a01 = jnp.full((L,), w01[i])
                    a10 = jnp.full((L,), w10[i])
                    a11 = jnp.full((L,), w11[i])
                    for j in range(C // L):
                        slj = pl.ds(j * L, L)
                        obuf_v[p, slj] = (r00_v[p, slj] * a00
                                          + r01_v[p, slj] * a01
                                          + r10_v[p, slj] * a10
                                          + r11_v[p, slj] * a11)
                return 0

            lax.fori_loop(0, PC // L, combine, 0)
            row0 = pl.multiple_of(patch * P + pc * PC, PC)
            pltpu.sync_copy(obuf_v, out_hbm.at[pl.ds(row0, PC), :])

        fire(0, ra_bufs, sem_a)

        def per_pair(k, _):
            pc = k * 2
            fire(pc + 1, rb_bufs, sem_b)
            drain(pc, ra_bufs, sem_a)
            consume(pc, ra_bufs)

            @pl.when(pc < NPC - 2)
            def _():
                fire(pc + 2, ra_bufs, sem_a)

            drain(pc + 1, rb_bufs, sem_b)
            consume(pc + 1, rb_bufs)
            return 0

        lax.fori_loop(0, NPC // 2, per_pair, 0)
        return 0

    lax.fori_loop(0, PPW, per_patch, 0)


def _build(interpret=False):
    mesh = plsc.VectorSubcoreMesh(core_axis_name="c", subcore_axis_name="s")
    return pl.kernel(
        _body,
        out_type=jax.ShapeDtypeStruct((NPB * P, C), jnp.float32),
        mesh=mesh,
        scratch_types=[
            pltpu.VMEM((NPB, L), jnp.float32),
            pltpu.VMEM((P,), jnp.float32),
            pltpu.VMEM((P,), jnp.float32),
            pltpu.VMEM((P,), jnp.float32),
            pltpu.VMEM((P,), jnp.float32),
            pltpu.VMEM((P,), jnp.int32),
            pltpu.VMEM((P,), jnp.int32),
            pltpu.VMEM((P,), jnp.int32),
            pltpu.VMEM((P,), jnp.int32),
            [pltpu.VMEM((PC, C), jnp.float32)] * 4,
            [pltpu.VMEM((PC, C), jnp.float32)] * 4,
            pltpu.VMEM((PC, C), jnp.float32),
            pltpu.SemaphoreType.DMA,
            pltpu.SemaphoreType.DMA,
        ],
        compiler_params=pltpu.CompilerParams(needs_layout_passes=False),
        interpret=interpret,
    )


def kernel(aer_feat, pose_uvr):
    theta = pose_uvr[..., 2]
    csu = jnp.stack(
        [jnp.cos(theta), -jnp.sin(theta), pose_uvr[..., 0], pose_uvr[..., 1]],
        axis=-1)
    csu = jnp.pad(csu, ((0, 0), (0, 0), (0, L - 4)))
    feat2 = aer_feat.reshape(B, C, H * W)
    sample = _build()
    relayout = _tc_transpose()
    outs = [sample(relayout(feat2[b]), csu[b]) for b in range(B)]
    out = jnp.stack(outs).reshape(B, N, P, C).transpose(0, 1, 3, 2)
    return out.reshape(B, N, C, HB, WB)


# speedup vs baseline: 2.3313x; 2.3313x over previous
import jax
import jax.numpy as jnp
from jax import lax
from jax.experimental import pallas as pl
from jax.experimental.pallas import tpu as pltpu
from jax.experimental.pallas import tpu_sc as plsc

B, C, H, W = 4, 128, 512, 512
N = 64
HB = WB = 32
P = HB * WB
NPATCH = B * N
NW = 32
PPW = NPATCH // NW
L = 16
PC = 64
NPC = P // PC


def _floor_vec(x):
    xi = x.astype(jnp.int32)
    xf = xi.astype(jnp.float32)
    neg = x < xf
    return jnp.where(neg, xi - 1, xi), jnp.where(neg, xf - 1.0, xf)


def _body(table_hbm, csu_hbm, out_hbm, csu_v,
          w00_v, w01_v, w10_v, w11_v, i00_v, i01_v, i10_v, i11_v,
          ra_bufs, rb_bufs, o_bufs, sem_a, sem_b, sem_oa, sem_ob):
    wid = lax.axis_index("s") * 2 + lax.axis_index("c")
    pltpu.sync_copy(csu_hbm, csu_v)
    idx_refs = (i00_v, i01_v, i10_v, i11_v)

    def fire(pc, bufs, sem):
        isl = pl.ds(pc * PC, PC)
        for iv, rv in zip(idx_refs, bufs):
            pltpu.async_copy(table_hbm.at[iv.at[isl]], rv, sem)

    def drain(pc, bufs, sem):
        isl = pl.ds(pc * PC, PC)
        for iv, rv in zip(idx_refs, bufs):
            pltpu.make_async_copy(table_hbm.at[iv.at[isl]], rv, sem).wait()

    def per_patch(t, _):
        patch = wid * PPW + t
        csrow = csu_v[patch, :]
        cs = csrow[0]
        sn = csrow[1]
        u = csrow[2]
        v = csrow[3]
        bhw = (patch >> 6) * (H * W)

        def grid_math(pg, _):
            pid = pg * L + lax.broadcasted_iota(jnp.int32, (L,), 0)
            hb = pid >> 5
            wb = pid & 31
            gu = (31 - hb).astype(jnp.float32)
            gv = (wb - 16).astype(jnp.float32)
            x = u + cs * gu - sn * gv
            y = v + sn * gu + cs * gv
            gx = (x + 0.5) * (2.0 / W) - 1.0
            gy = (y + 0.5) * (2.0 / H) - 1.0
            pv = (jnp.abs(gx) < 1.0) & (jnp.abs(gy) < 1.0)
            x0i, x0f = _floor_vec(x)
            y0i, y0f = _floor_vec(y)
            wx1 = x - x0f
            wx0 = 1.0 - wx1
            wy1 = y - y0f
            wy0 = 1.0 - wy1
            vx0 = (x0f >= 0.0) & (x0f <= W - 1.0)
            vx1 = (x0f + 1.0 >= 0.0) & (x0f + 1.0 <= W - 1.0)
            vy0 = (y0f >= 0.0) & (y0f <= H - 1.0)
            vy1 = (y0f + 1.0 >= 0.0) & (y0f + 1.0 <= H - 1.0)
            x0c = jnp.clip(x0i, 0, W - 1)
            x1c = jnp.clip(x0i + 1, 0, W - 1)
            y0r = jnp.clip(y0i, 0, H - 1) * W + bhw
            y1r = jnp.clip(y0i + 1, 0, H - 1) * W + bhw
            sl = pl.ds(pg * L, L)
            w00_v[sl] = jnp.where(vx0 & vy0 & pv, wx0 * wy0, 0.0)
            w01_v[sl] = jnp.where(vx1 & vy0 & pv, wx1 * wy0, 0.0)
            w10_v[sl] = jnp.where(vx0 & vy1 & pv, wx0 * wy1, 0.0)
            w11_v[sl] = jnp.where(vx1 & vy1 & pv, wx1 * wy1, 0.0)
            i00_v[sl] = y0r + x0c
            i01_v[sl] = y0r + x1c
            i10_v[sl] = y1r + x0c
            i11_v[sl] = y1r + x1c
            return 0

        lax.fori_loop(0, P // L, grid_math, 0)

        def out_row(pc):
            return pl.ds(pl.multiple_of(patch * P + pc * PC, PC), PC)

        def consume(pc, bufs, obuf_v, sem_o):
            r00_v, r01_v, r10_v, r11_v = bufs

            @pl.when(pc >= 2)
            def _():
                pltpu.make_async_copy(
                    obuf_v, out_hbm.at[out_row(pc), :], sem_o).wait()

            def combine(pg, _):
                psl = pl.ds(pc * PC + pg * L, L)
                w00 = w00_v[psl]
                w01 = w01_v[psl]
                w10 = w10_v[psl]
                w11 = w11_v[psl]
                for i in range(L):
                    p = pg * L + i
                    a00 = jnp.full((L,), w00[i])
                    a01 = jnp.full((L,), w01[i])
                    a10 = jnp.full((L,), w10[i])
                    a11 = jnp.full((L,), w11[i])
                    for j in range(C // L):
                        slj = pl.ds(j * L, L)
                        obuf_v[p, slj] = (r00_v[p, slj] * a00
                                          + r01_v[p, slj] * a01
                                          + r10_v[p, slj] * a10
                                          + r11_v[p, slj] * a11)
                return 0

            lax.fori_loop(0, PC // L, combine, 0)
            pltpu.async_copy(obuf_v, out_hbm.at[out_row(pc), :], sem_o)

        fire(0, ra_bufs, sem_a)

        def per_pair(k, _):
            pc = k * 2
            fire(pc + 1, rb_bufs, sem_b)
            drain(pc, ra_bufs, sem_a)
            consume(pc, ra_bufs, o_bufs[0], sem_oa)

            @pl.when(pc < NPC - 2)
            def _():
                fire(pc + 2, ra_bufs, sem_a)

            drain(pc + 1, rb_bufs, sem_b)
            consume(pc + 1, rb_bufs, o_bufs[1], sem_ob)
            return 0

        lax.fori_loop(0, NPC // 2, per_pair, 0)
        pltpu.make_async_copy(
            o_bufs[0], out_hbm.at[out_row(NPC - 2), :], sem_oa).wait()
        pltpu.make_async_copy(
            o_bufs[1], out_hbm.at[out_row(NPC - 1), :], sem_ob).wait()
        return 0

    lax.fori_loop(0, PPW, per_patch, 0)


def _build(interpret=False):
    mesh = plsc.VectorSubcoreMesh(core_axis_name="c", subcore_axis_name="s")
    return pl.kernel(
        _body,
        out_type=jax.ShapeDtypeStruct((NPATCH * P, C), jnp.float32),
        mesh=mesh,
        scratch_types=[
            pltpu.VMEM((NPATCH, L), jnp.float32),
            pltpu.VMEM((P,), jnp.float32),
            pltpu.VMEM((P,), jnp.float32),
            pltpu.VMEM((P,), jnp.float32),
            pltpu.VMEM((P,), jnp.float32),
            pltpu.VMEM((P,), jnp.int32),
            pltpu.VMEM((P,), jnp.int32),
            pltpu.VMEM((P,), jnp.int32),
            pltpu.VMEM((P,), jnp.int32),
            [pltpu.VMEM((PC, C), jnp.float32)] * 4,
            [pltpu.VMEM((PC, C), jnp.float32)] * 4,
            [pltpu.VMEM((PC, C), jnp.float32)] * 2,
            pltpu.SemaphoreType.DMA,
            pltpu.SemaphoreType.DMA,
            pltpu.SemaphoreType.DMA,
            pltpu.SemaphoreType.DMA,
        ],
        compiler_params=pltpu.CompilerParams(needs_layout_passes=False),
        interpret=interpret,
    )


def kernel(aer_feat, pose_uvr):
    theta = pose_uvr[..., 2]
    csu = jnp.stack(
        [jnp.cos(theta), -jnp.sin(theta), pose_uvr[..., 0], pose_uvr[..., 1]],
        axis=-1).reshape(NPATCH, 4)
    csu = jnp.pad(csu, ((0, 0), (0, L - 4)))
    table = aer_feat.transpose(0, 2, 3, 1).reshape(B * H * W, C)
    out = _build()(table, csu)
    out = out.reshape(B, N, P, C).transpose(0, 1, 3, 2)
    return out.reshape(B, N, C, HB, WB)
